# baseline (speedup 1.0000x reference)
import functools

import numpy as np
import jax
import jax.numpy as jnp
from jax import lax
from jax.experimental import pallas as pl
from jax.experimental.pallas import tpu as pltpu
from jax.experimental.pallas import tpu_sc as plsc

_B = 8
_HH = 224
_W = 224
_D = 96
_DP = 128
_N = _B * _HH * _W
_NT = _B * _HH
_NC = 2
_NS = 16
_NW = _NC * _NS
_TPW = _NT // _NW
_RW = _N // _NW
_L = 16


def _compute_permutation():
    with jax.set_mesh(None), jax.ensure_compile_time_eval():
        p = jax.random.permutation(jax.random.key(42), _N)
        return np.asarray(p, dtype=np.int32)


_PERM = _compute_permutation()


def _sc_stage(feat_t, mask_t):
    mesh = plsc.VectorSubcoreMesh(core_axis_name="c", subcore_axis_name="s")

    @functools.partial(
        pl.kernel,
        out_type=jax.ShapeDtypeStruct((_N, _DP), jnp.float32),
        mesh=mesh,
        compiler_params=pltpu.CompilerParams(needs_layout_passes=False),
        scratch_types=[
            pltpu.VMEM((_D, _W), jnp.float32),
            pltpu.VMEM((1, _W), jnp.float32),
            pltpu.VMEM((_W, _DP), jnp.float32),
        ],
    )
    def k(feat_hbm, mask_hbm, h_hbm, fbuf, mbuf, hbuf):
        wid = lax.axis_index("c") * _NS + lax.axis_index("s")
        iotas = [lax.iota(jnp.int32, _L) + c * _L for c in range(_D // _L)]

        def tile_body(kk, carry):
            t = wid * _TPW + kk
            b = t // _HH
            hh = t % _HH
            pltpu.sync_copy(feat_hbm.at[b, hh], fbuf)
            pltpu.sync_copy(mask_hbm.at[b, hh], mbuf)

            def px_body(w, c2):
                ws = jnp.full((_L,), w, dtype=jnp.int32)
                mv = plsc.load_gather(mbuf, [jnp.zeros((_L,), jnp.int32), ws])
                wm = 1.0 - mv
                for ci in iotas:
                    v = plsc.load_gather(fbuf, [ci, ws])
                    plsc.store_scatter(hbuf, [ws, ci], wm * v)
                return c2

            lax.fori_loop(0, _W, px_body, 0)
            pltpu.sync_copy(hbuf, h_hbm.at[pl.ds(t * _W, _W)])
            return carry

        lax.fori_loop(0, _TPW, tile_body, 0)

    return k(feat_t, mask_t)


def _sc_gather_combine(h, feat_t, mask_t, perm):
    mesh = plsc.VectorSubcoreMesh(core_axis_name="c", subcore_axis_name="s")

    @functools.partial(
        pl.kernel,
        out_type=jax.ShapeDtypeStruct((_B, _HH, _D, _W), jnp.float32),
        mesh=mesh,
        compiler_params=pltpu.CompilerParams(needs_layout_passes=False),
        scratch_types=[
            pltpu.VMEM((_RW,), jnp.int32),
            pltpu.VMEM((_W, _DP), jnp.float32),
            pltpu.VMEM((_D, _W), jnp.float32),
            pltpu.VMEM((1, _W), jnp.float32),
            pltpu.VMEM((_D, _W), jnp.float32),
            pltpu.SemaphoreType.DMA,
            pltpu.SemaphoreType.DMA,
            pltpu.SemaphoreType.DMA,
        ],
    )
    def k(h_hbm, feat_hbm, mask_hbm, perm_hbm, out_hbm,
          idx_v, gbuf, fbuf, mbuf, obuf, sem_g, sem_f, sem_m):
        wid = lax.axis_index("c") * _NS + lax.axis_index("s")
        rbase = wid * _RW
        pltpu.sync_copy(perm_hbm.at[pl.ds(rbase, _RW)], idx_v)
        iotas = [lax.iota(jnp.int32, _L) + c * _L for c in range(_D // _L)]

        def tile_body(kk, carry):
            t = wid * _TPW + kk
            b = t // _HH
            hh = t % _HH
            r0 = kk * _W
            cg0 = pltpu.async_copy(
                h_hbm.at[idx_v.at[pl.ds(r0, _W // 2)]],
                gbuf.at[pl.ds(0, _W // 2)], sem_g)
            cg1 = pltpu.async_copy(
                h_hbm.at[idx_v.at[pl.ds(r0 + _W // 2, _W // 2)]],
                gbuf.at[pl.ds(_W // 2, _W // 2)], sem_g)
            cf = pltpu.async_copy(feat_hbm.at[b, hh], fbuf, sem_f)
            cm = pltpu.async_copy(mask_hbm.at[b, hh], mbuf, sem_m)
            cg0.wait()
            cg1.wait()
            cf.wait()
            cm.wait()

            def px_body(w, c2):
                ws = jnp.full((_L,), w, dtype=jnp.int32)
                mv = plsc.load_gather(mbuf, [jnp.zeros((_L,), jnp.int32), ws])
                wm = 1.0 - mv
                for ci in iotas:
                    gv = plsc.load_gather(gbuf, [ws, ci])
                    fv = plsc.load_gather(fbuf, [ci, ws])
                    plsc.store_scatter(obuf, [ci, ws], mv * fv + wm * gv)
                return c2

            lax.fori_loop(0, _W, px_body, 0)
            pltpu.sync_copy(obuf, out_hbm.at[b, hh])
            return carry

        lax.fori_loop(0, _TPW, tile_body, 0)

    return k(h, feat_t, mask_t, perm)


def kernel(feat, mask):
    feat_t = feat.transpose(0, 1, 3, 2)
    mask_t = mask.transpose(0, 1, 3, 2)
    perm = jnp.asarray(_PERM)
    h = _sc_stage(feat_t, mask_t)
    out_t = _sc_gather_combine(h, feat_t, mask_t, perm)
    return out_t.transpose(0, 1, 3, 2)


# speedup vs baseline: 1.3530x; 1.3530x over previous
import functools

import numpy as np
import jax
import jax.numpy as jnp
from jax import lax
from jax.experimental import pallas as pl
from jax.experimental.pallas import tpu as pltpu
from jax.experimental.pallas import tpu_sc as plsc

_B = 8
_HH = 224
_W = 224
_D = 96
_DP = 128
_N = _B * _HH * _W
_NT = _B * _HH
_NC = 2
_NS = 16
_NW = _NC * _NS
_CH = 128
_RW = _N // _NW
_NCH_W = _RW // _CH


def _compute_permutation():
    with jax.set_mesh(None), jax.ensure_compile_time_eval():
        p = jax.random.permutation(jax.random.key(42), _N)
        return np.asarray(p, dtype=np.int32)


_PERM = _compute_permutation()


def _tc_stage(feat_t, mask_t):
    def body(f_ref, m_ref, h_ref):
        f = f_ref[0, 0]
        m = m_ref[0, 0]
        bg = (1.0 - m) * f
        h_ref[:, : _D] = jnp.transpose(bg, (1, 0))

    return pl.pallas_call(
        body,
        grid=(_NT,),
        in_specs=[
            pl.BlockSpec((1, 1, _D, _W), lambda i: (i // _HH, i % _HH, 0, 0)),
            pl.BlockSpec((1, 1, 1, _W), lambda i: (i // _HH, i % _HH, 0, 0)),
        ],
        out_specs=pl.BlockSpec((_W, _DP), lambda i: (i, 0)),
        out_shape=jax.ShapeDtypeStruct((_N, _DP), jnp.float32),
    )(feat_t, mask_t)


def _sc_gather(h, perm):
    mesh = plsc.VectorSubcoreMesh(core_axis_name="c", subcore_axis_name="s")

    @functools.partial(
        pl.kernel,
        out_type=jax.ShapeDtypeStruct((_N, _DP), jnp.float32),
        mesh=mesh,
        scratch_types=[
            pltpu.VMEM((_RW,), jnp.int32),
            pltpu.VMEM((_CH, _DP), jnp.float32),
            pltpu.VMEM((_CH, _DP), jnp.float32),
            pltpu.SemaphoreType.DMA,
            pltpu.SemaphoreType.DMA,
        ],
    )
    def k(h_hbm, perm_hbm, g_hbm, idx_v, buf0, buf1, sem0, sem1):
        wid = lax.axis_index("c") * _NS + lax.axis_index("s")
        rbase = wid * _RW
        pltpu.sync_copy(perm_hbm.at[pl.ds(rbase, _RW)], idx_v)

        def body(jj, carry):
            j0 = jj * 2
            idx0 = idx_v.at[pl.ds(j0 * _CH, _CH)]
            idx1 = idx_v.at[pl.ds((j0 + 1) * _CH, _CH)]
            cp0 = pltpu.async_copy(h_hbm.at[idx0], buf0, sem0)
            cp1 = pltpu.async_copy(h_hbm.at[idx1], buf1, sem1)
            row0 = rbase + j0 * _CH
            cp0.wait()
            pltpu.sync_copy(buf0, g_hbm.at[pl.ds(row0, _CH)])
            cp1.wait()
            pltpu.sync_copy(buf1, g_hbm.at[pl.ds(row0 + _CH, _CH)])
            return carry

        lax.fori_loop(0, _NCH_W // 2, body, 0)

    return k(h, perm)


def _tc_combine(feat_t, mask_t, g):
    def body(f_ref, m_ref, g_ref, o_ref):
        f = f_ref[0, 0]
        m = m_ref[0, 0]
        gt = jnp.transpose(g_ref[:, : _D], (1, 0))
        o_ref[0, 0] = m * f + (1.0 - m) * gt

    return pl.pallas_call(
        body,
        grid=(_NT,),
        in_specs=[
            pl.BlockSpec((1, 1, _D, _W), lambda i: (i // _HH, i % _HH, 0, 0)),
            pl.BlockSpec((1, 1, 1, _W), lambda i: (i // _HH, i % _HH, 0, 0)),
            pl.BlockSpec((_W, _DP), lambda i: (i, 0)),
        ],
        out_specs=pl.BlockSpec((1, 1, _D, _W), lambda i: (i // _HH, i % _HH, 0, 0)),
        out_shape=jax.ShapeDtypeStruct((_B, _HH, _D, _W), jnp.float32),
    )(feat_t, mask_t, g)


def kernel(feat, mask):
    feat_t = feat.transpose(0, 1, 3, 2)
    mask_t = mask.transpose(0, 1, 3, 2)
    perm = jnp.asarray(_PERM)
    h = _tc_stage(feat_t, mask_t)
    g = _sc_gather(h, perm)
    out_t = _tc_combine(feat_t, mask_t, g)
    return out_t.transpose(0, 1, 3, 2)


# speedup vs baseline: 5.6758x; 4.1951x over previous
import functools

import numpy as np
import jax
import jax.numpy as jnp
from jax import lax
from jax.experimental import pallas as pl
from jax.experimental.pallas import tpu as pltpu
from jax.experimental.pallas import tpu_sc as plsc

_B = 8
_HH = 224
_W = 224
_D = 96
_DP = 128
_N = _B * _HH * _W
_NT = _B * _HH
_NC = 2
_NS = 16
_NW = _NC * _NS
_CH = 128
_RW = _N // _NW
_NCH_W = _RW // _CH


def _compute_permutation():
    with jax.set_mesh(None), jax.ensure_compile_time_eval():
        p = jax.random.permutation(jax.random.key(42), _N)
        return np.asarray(p, dtype=np.int32)


_PERM = _compute_permutation()


_TB = 16
_NB = _HH // _TB


def _tc_stage(feat_t, mask_t):
    def body(f_ref, m_ref, h_ref):
        f = f_ref[0]
        m = m_ref[0]
        bg = (1.0 - m) * f
        t = jnp.transpose(bg, (0, 2, 1))
        h_ref[:, : _D] = t.reshape(_TB * _W, _D)

    return pl.pallas_call(
        body,
        grid=(_NT // _TB,),
        in_specs=[
            pl.BlockSpec((1, _TB, _D, _W), lambda i: (i // _NB, i % _NB, 0, 0)),
            pl.BlockSpec((1, _TB, 1, _W), lambda i: (i // _NB, i % _NB, 0, 0)),
        ],
        out_specs=pl.BlockSpec((_TB * _W, _DP), lambda i: (i, 0)),
        out_shape=jax.ShapeDtypeStruct((_N, _DP), jnp.float32),
    )(feat_t, mask_t)


def _sc_gather(h, perm):
    mesh = plsc.VectorSubcoreMesh(core_axis_name="c", subcore_axis_name="s")

    @functools.partial(
        pl.kernel,
        out_type=jax.ShapeDtypeStruct((_N, _DP), jnp.float32),
        mesh=mesh,
        scratch_types=[
            pltpu.VMEM((_RW,), jnp.int32),
            pltpu.VMEM((_CH, _DP), jnp.float32),
            pltpu.VMEM((_CH, _DP), jnp.float32),
            pltpu.SemaphoreType.DMA,
            pltpu.SemaphoreType.DMA,
        ],
    )
    def k(h_hbm, perm_hbm, g_hbm, idx_v, buf0, buf1, sem0, sem1):
        wid = lax.axis_index("c") * _NS + lax.axis_index("s")
        rbase = wid * _RW
        pltpu.sync_copy(perm_hbm.at[pl.ds(rbase, _RW)], idx_v)

        def body(jj, carry):
            j0 = jj * 2
            idx0 = idx_v.at[pl.ds(j0 * _CH, _CH)]
            idx1 = idx_v.at[pl.ds((j0 + 1) * _CH, _CH)]
            cp0 = pltpu.async_copy(h_hbm.at[idx0], buf0, sem0)
            cp1 = pltpu.async_copy(h_hbm.at[idx1], buf1, sem1)
            row0 = rbase + j0 * _CH
            cp0.wait()
            pltpu.sync_copy(buf0, g_hbm.at[pl.ds(row0, _CH)])
            cp1.wait()
            pltpu.sync_copy(buf1, g_hbm.at[pl.ds(row0 + _CH, _CH)])
            return carry

        lax.fori_loop(0, _NCH_W // 2, body, 0)

    return k(h, perm)


def _tc_combine(feat_t, mask_t, g):
    def body(f_ref, m_ref, g_ref, o_ref):
        f = f_ref[0]
        m = m_ref[0]
        gr = g_ref[:, : _D].reshape(_TB, _W, _D)
        gt = jnp.transpose(gr, (0, 2, 1))
        o_ref[0] = m * f + (1.0 - m) * gt

    return pl.pallas_call(
        body,
        grid=(_NT // _TB,),
        in_specs=[
            pl.BlockSpec((1, _TB, _D, _W), lambda i: (i // _NB, i % _NB, 0, 0)),
            pl.BlockSpec((1, _TB, 1, _W), lambda i: (i // _NB, i % _NB, 0, 0)),
            pl.BlockSpec((_TB * _W, _DP), lambda i: (i, 0)),
        ],
        out_specs=pl.BlockSpec((1, _TB, _D, _W), lambda i: (i // _NB, i % _NB, 0, 0)),
        out_shape=jax.ShapeDtypeStruct((_B, _HH, _D, _W), jnp.float32),
    )(feat_t, mask_t, g)


def kernel(feat, mask):
    feat_t = feat.transpose(0, 1, 3, 2)
    mask_t = mask.transpose(0, 1, 3, 2)
    perm = jnp.asarray(_PERM)
    h = _tc_stage(feat_t, mask_t)
    g = _sc_gather(h, perm)
    out_t = _tc_combine(feat_t, mask_t, g)
    return out_t.transpose(0, 1, 3, 2)
